# baseline (device time: 110866 ns/iter reference)
import jax
import jax.numpy as jnp
from jax import lax
from jax.experimental import pallas as pl
from jax.experimental.pallas import tpu as pltpu

N_DEV = 8
SQ = 2048
SKV_LOCAL = 2048
HQ = 8
DH = 128
DM = HQ * DH
QBLK = 256
CBLK = 128
NBLK = SQ // QBLK
NCH = SQ // CBLK
SCALE = 0.08838834764831843

CHILDREN = {0: (1, 3, 4), 1: (2, 5), 3: (7,), 4: (6,)}
PARENT = {1: 0, 2: 1, 3: 0, 4: 0, 5: 1, 6: 4, 7: 3}
PEERS = {
    0: (1, 3, 4), 1: (0, 2, 5), 2: (1,), 3: (0, 7),
    4: (0, 6), 5: (1,), 6: (4,), 7: (3,),
}


def kernel(x, Wq, K_ext, V_ext, Wo):
    x2 = x[0]
    k2 = K_ext[0].reshape(SKV_LOCAL, DM).astype(jnp.bfloat16)
    v2 = V_ext[0].reshape(SKV_LOCAL, DM).astype(jnp.bfloat16)

    def body(x_ref, wq_ref, k_ref, v_ref, wo_ref, out_ref,
             ctx_ref, wqbf_ref, wobf_ref, send_sems, recv_sems):
        my = lax.axis_index("i")

        wobf_ref[...] = wo_ref[...].astype(jnp.bfloat16)

        def chunk_copy(c, target, d):
            return pltpu.make_async_remote_copy(
                src_ref=ctx_ref.at[pl.ds(c * CBLK, CBLK)],
                dst_ref=ctx_ref.at[pl.ds(c * CBLK, CBLK)],
                send_sem=send_sems.at[c, d],
                recv_sem=recv_sems.at[c],
                device_id=(target,), device_id_type=pl.DeviceIdType.MESH,
            )

        def project(c, rows):
            out_ref[0, c * rows:(c + 1) * rows, :] = jnp.dot(
                ctx_ref[c * rows:(c + 1) * rows, :], wobf_ref[...],
                preferred_element_type=jnp.float32,
            ).astype(jnp.bfloat16)

        def barrier(peers):
            bsem = pltpu.get_barrier_semaphore()
            for pr in peers:
                pl.semaphore_signal(
                    bsem, inc=1,
                    device_id=(pr,), device_id_type=pl.DeviceIdType.MESH,
                )
            pl.semaphore_wait(bsem, len(peers))

        @pl.when(my == 0)
        def _():
            barrier(PEERS[0])
            wqbf_ref[...] = wq_ref[...].astype(jnp.bfloat16)
            rd = lax.broadcasted_iota(jnp.int32, (QBLK, QBLK), 0) // 64
            cd = lax.broadcasted_iota(jnp.int32, (QBLK, QBLK), 1) // 64
            dmask = cd <= rd
            for b in range(NBLK):
                off = b * QBLK
                xb = x_ref[off:off + QBLK, :].astype(jnp.bfloat16)
                qb = jnp.dot(
                    xb, wqbf_ref[...], preferred_element_type=jnp.float32
                ).astype(jnp.bfloat16)
                for h in range(HQ):
                    qh = qb[:, h * DH:(h + 1) * DH]
                    kd = k_ref[off:off + QBLK, h * DH:(h + 1) * DH]
                    vd = v_ref[off:off + QBLK, h * DH:(h + 1) * DH]
                    sd = lax.dot_general(
                        qh, kd, (((1,), (1,)), ((), ())),
                        preferred_element_type=jnp.float32,
                    ) * SCALE
                    sd = jnp.where(dmask, sd, -1e9)
                    pd = jnp.exp(sd)
                    l = jnp.sum(pd, axis=-1, keepdims=True)
                    cun = jnp.dot(
                        pd.astype(jnp.bfloat16), vd,
                        preferred_element_type=jnp.float32,
                    )
                    if b > 0:
                        kf = k_ref[:off, h * DH:(h + 1) * DH]
                        vf = v_ref[:off, h * DH:(h + 1) * DH]
                        sf = lax.dot_general(
                            qh, kf, (((1,), (1,)), ((), ())),
                            preferred_element_type=jnp.float32,
                        ) * SCALE
                        pf = jnp.exp(sf)
                        l = l + jnp.sum(pf, axis=-1, keepdims=True)
                        cun = cun + jnp.dot(
                            pf.astype(jnp.bfloat16), vf,
                            preferred_element_type=jnp.float32,
                        )
                    ctx_ref[off:off + QBLK, h * DH:(h + 1) * DH] = (
                        (cun * (1.0 / l)).astype(jnp.bfloat16)
                    )
                for c in (2 * b, 2 * b + 1):
                    for d, tgt in enumerate(CHILDREN[0]):
                        chunk_copy(c, tgt, d).start()
            for b in range(NBLK):
                project(b, QBLK)
            for c in range(NCH):
                for d, tgt in enumerate(CHILDREN[0]):
                    chunk_copy(c, tgt, d).wait_send()

        for pos in range(1, N_DEV):

            @pl.when(my == pos)
            def _(pos=pos):
                barrier(PEERS[pos])
                for c in range(NCH):
                    chunk_copy(c, PARENT[pos], 0).wait_recv()
                    for d, tgt in enumerate(CHILDREN.get(pos, ())):
                        chunk_copy(c, tgt, d).start()
                    project(c, CBLK)
                for c in range(NCH):
                    for d, tgt in enumerate(CHILDREN.get(pos, ())):
                        chunk_copy(c, tgt, d).wait_send()

    out = pl.pallas_call(
        body,
        out_shape=jax.ShapeDtypeStruct((1, SQ, DM), jnp.bfloat16),
        in_specs=[pl.BlockSpec(memory_space=pltpu.VMEM)] * 5,
        out_specs=pl.BlockSpec(memory_space=pltpu.VMEM),
        scratch_shapes=[
            pltpu.VMEM((SQ, DM), jnp.bfloat16),
            pltpu.VMEM((DM, DM), jnp.bfloat16),
            pltpu.VMEM((DM, DM), jnp.bfloat16),
            pltpu.SemaphoreType.DMA((NCH, 3)),
            pltpu.SemaphoreType.DMA((NCH,)),
        ],
        compiler_params=pltpu.CompilerParams(
            collective_id=0, vmem_limit_bytes=64 * 1024 * 1024
        ),
    )(x2, Wq, k2, v2, Wo)
    return out


# device time: 101162 ns/iter; 1.0959x vs baseline; 1.0959x over previous
import jax
import jax.numpy as jnp
from jax import lax
from jax.experimental import pallas as pl
from jax.experimental.pallas import tpu as pltpu

N_DEV = 8
SQ = 2048
SKV_LOCAL = 2048
HQ = 8
DH = 128
DM = HQ * DH
QBLK = 256
CBLK = 256
NBLK = SQ // QBLK
NCH = SQ // CBLK
SCALE = 0.08838834764831843

CHILDREN = {0: (1, 3, 4), 1: (2, 5), 3: (7,), 4: (6,)}
PARENT = {1: 0, 2: 1, 3: 0, 4: 0, 5: 1, 6: 4, 7: 3}
PEERS = {
    0: (1, 3, 4), 1: (0, 2, 5), 2: (1,), 3: (0, 7),
    4: (0, 6), 5: (1,), 6: (4,), 7: (3,),
}


def kernel(x, Wq, K_ext, V_ext, Wo):
    x2 = x[0]
    k2 = K_ext[0].reshape(SKV_LOCAL, DM).astype(jnp.bfloat16)
    v2 = V_ext[0].reshape(SKV_LOCAL, DM).astype(jnp.bfloat16)

    def body(x_ref, wq_ref, k_ref, v_ref, wo_ref, out_ref,
             ctx_ref, wqbf_ref, wobf_ref, send_sems, recv_sems):
        my = lax.axis_index("i")

        wobf_ref[...] = wo_ref[...].astype(jnp.bfloat16)

        def chunk_copy(c, target, d):
            return pltpu.make_async_remote_copy(
                src_ref=ctx_ref.at[pl.ds(c * CBLK, CBLK)],
                dst_ref=ctx_ref.at[pl.ds(c * CBLK, CBLK)],
                send_sem=send_sems.at[c, d],
                recv_sem=recv_sems.at[c],
                device_id=(target,), device_id_type=pl.DeviceIdType.MESH,
            )

        def project(c, rows):
            out_ref[0, c * rows:(c + 1) * rows, :] = jnp.dot(
                ctx_ref[c * rows:(c + 1) * rows, :], wobf_ref[...],
                preferred_element_type=jnp.float32,
            ).astype(jnp.bfloat16)

        def barrier(peers):
            bsem = pltpu.get_barrier_semaphore()
            for pr in peers:
                pl.semaphore_signal(
                    bsem, inc=1,
                    device_id=(pr,), device_id_type=pl.DeviceIdType.MESH,
                )
            pl.semaphore_wait(bsem, len(peers))

        @pl.when(my == 0)
        def _():
            barrier(PEERS[0])
            wqbf_ref[...] = wq_ref[...].astype(jnp.bfloat16)
            rd = lax.broadcasted_iota(jnp.int32, (QBLK, QBLK), 0) // 64
            cd = lax.broadcasted_iota(jnp.int32, (QBLK, QBLK), 1) // 64
            dmask = cd <= rd
            for b in range(NBLK):
                off = b * QBLK
                xb = x_ref[off:off + QBLK, :].astype(jnp.bfloat16)
                qb = jnp.dot(
                    xb, wqbf_ref[...], preferred_element_type=jnp.float32
                ).astype(jnp.bfloat16)
                for h in range(HQ):
                    qh = qb[:, h * DH:(h + 1) * DH]
                    kd = k_ref[off:off + QBLK, h * DH:(h + 1) * DH]
                    vd = v_ref[off:off + QBLK, h * DH:(h + 1) * DH]
                    sd = lax.dot_general(
                        qh, kd, (((1,), (1,)), ((), ())),
                        preferred_element_type=jnp.float32,
                    ) * SCALE
                    sd = jnp.where(dmask, sd, -1e9)
                    pd = jnp.exp(sd)
                    l = jnp.sum(pd, axis=-1, keepdims=True)
                    cun = jnp.dot(
                        pd.astype(jnp.bfloat16), vd,
                        preferred_element_type=jnp.float32,
                    )
                    if b > 0:
                        kf = k_ref[:off, h * DH:(h + 1) * DH]
                        vf = v_ref[:off, h * DH:(h + 1) * DH]
                        sf = lax.dot_general(
                            qh, kf, (((1,), (1,)), ((), ())),
                            preferred_element_type=jnp.float32,
                        ) * SCALE
                        pf = jnp.exp(sf)
                        l = l + jnp.sum(pf, axis=-1, keepdims=True)
                        cun = cun + jnp.dot(
                            pf.astype(jnp.bfloat16), vf,
                            preferred_element_type=jnp.float32,
                        )
                    ctx_ref[off:off + QBLK, h * DH:(h + 1) * DH] = (
                        (cun * (1.0 / l)).astype(jnp.bfloat16)
                    )
                for d, tgt in enumerate(CHILDREN[0]):
                    chunk_copy(b, tgt, d).start()
            for b in range(NBLK):
                project(b, QBLK)
            for c in range(NCH):
                for d, tgt in enumerate(CHILDREN[0]):
                    chunk_copy(c, tgt, d).wait_send()

        for pos in range(1, N_DEV):

            @pl.when(my == pos)
            def _(pos=pos):
                barrier(PEERS[pos])
                for c in range(NCH):
                    chunk_copy(c, PARENT[pos], 0).wait_recv()
                    for d, tgt in enumerate(CHILDREN.get(pos, ())):
                        chunk_copy(c, tgt, d).start()
                    project(c, CBLK)
                for c in range(NCH):
                    for d, tgt in enumerate(CHILDREN.get(pos, ())):
                        chunk_copy(c, tgt, d).wait_send()

    out = pl.pallas_call(
        body,
        out_shape=jax.ShapeDtypeStruct((1, SQ, DM), jnp.bfloat16),
        in_specs=[pl.BlockSpec(memory_space=pltpu.VMEM)] * 5,
        out_specs=pl.BlockSpec(memory_space=pltpu.VMEM),
        scratch_shapes=[
            pltpu.VMEM((SQ, DM), jnp.bfloat16),
            pltpu.VMEM((DM, DM), jnp.bfloat16),
            pltpu.VMEM((DM, DM), jnp.bfloat16),
            pltpu.SemaphoreType.DMA((NCH, 3)),
            pltpu.SemaphoreType.DMA((NCH,)),
        ],
        compiler_params=pltpu.CompilerParams(
            collective_id=0, vmem_limit_bytes=64 * 1024 * 1024
        ),
    )(x2, Wq, k2, v2, Wo)
    return out
